# baseline (speedup 1.0000x reference)
import functools

import jax
import jax.numpy as jnp
from jax import lax
from jax.experimental import pallas as pl
from jax.experimental.pallas import tpu as pltpu
from jax.experimental.pallas import tpu_sc as plsc

DIM = 128
LANES = 16
CHUNK = 64
NSTEP = DIM // LANES

_PERM_DNUMS = lax.GatherDimensionNumbers(
    offset_dims=(), collapsed_slice_dims=(0,), start_index_map=(0,))


def _lane_perm(x, idx):
    return lax.gather(x, idx[:, None], _PERM_DNUMS, (1,),
                      mode=lax.GatherScatterMode.PROMISE_IN_BOUNDS)


NSLOT = 4


def _bpr_body(user_hbm, item_hbm, w_hbm, h_hbm, out_hbm,
              uidx_v, iidx_v, wbufs, hbufs, out_v, sems, *,
              num_cores, bpw):
    wid = lax.axis_index("s") * num_cores + lax.axis_index("c")
    nchunk = bpw // CHUNK
    nsuper = nchunk // NSLOT
    base = wid * bpw

    pltpu.sync_copy(user_hbm.at[pl.ds(base, bpw)], uidx_v)
    pltpu.sync_copy(item_hbm.at[pl.ds(base, bpw)], iidx_v)

    lane_iota = lax.iota(jnp.int32, LANES)

    def fire(c, j):
        off = pl.multiple_of(c * CHUNK, CHUNK)
        pltpu.async_copy(w_hbm.at[uidx_v.at[pl.ds(off, CHUNK)]],
                         wbufs.at[j], sems.at[0, j])
        pltpu.async_copy(h_hbm.at[iidx_v.at[pl.ds(off, CHUNK)]],
                         hbufs.at[j], sems.at[1, j])

    def wait(j):
        pltpu.make_async_copy(w_hbm.at[uidx_v.at[pl.ds(0, CHUNK)]],
                              wbufs.at[j], sems.at[0, j]).wait()
        pltpu.make_async_copy(h_hbm.at[iidx_v.at[pl.ds(0, CHUNK)]],
                              hbufs.at[j], sems.at[1, j]).wait()

    def compute(c, j):
        wbuf = wbufs.at[j]
        hbuf = hbufs.at[j]
        def row_body(row, res):
            acc = (wbuf[row, pl.ds(0, LANES)]
                   * hbuf[row, pl.ds(0, LANES)])
            for k in range(1, NSTEP):
                acc = acc + (wbuf[row, pl.ds(k * LANES, LANES)]
                             * hbuf[row, pl.ds(k * LANES, LANES)])
            for sh in (8, 4, 2, 1):
                acc = acc + _lane_perm(acc, lane_iota ^ sh)
            res = jnp.where(lane_iota == (row & (LANES - 1)), acc, res)
            @pl.when((row & (LANES - 1)) == LANES - 1)
            def _():
                out_v[pl.ds(c * CHUNK + (row & -LANES), LANES)] = res
            return res

        lax.fori_loop(0, CHUNK, row_body,
                      jnp.zeros((LANES,), jnp.float32), unroll=8)

    for j in range(NSLOT):
        fire(j, j)

    def super_body(t, carry):
        c0 = t * NSLOT
        for j in range(NSLOT):
            wait(j)
            compute(c0 + j, j)

            @pl.when(t + 1 < nsuper)
            def _(j=j):
                fire(c0 + j + NSLOT, j)
        return carry

    lax.fori_loop(0, nsuper, super_body, 0)

    pltpu.sync_copy(out_v, out_hbm.at[pl.ds(base, bpw)])


def kernel(user_idx, item_idx, W, H):
    batch = user_idx.shape[0]
    info = plsc.get_sparse_core_info()
    num_workers = info.num_cores * info.num_subcores
    bpw = batch // num_workers

    mesh = plsc.VectorSubcoreMesh(core_axis_name="c", subcore_axis_name="s")
    body = functools.partial(_bpr_body, num_cores=info.num_cores, bpw=bpw)

    y = pl.kernel(
        body,
        mesh=mesh,
        out_type=jax.ShapeDtypeStruct((batch,), jnp.float32),
        scratch_types=[
            pltpu.VMEM((bpw,), jnp.int32),
            pltpu.VMEM((bpw,), jnp.int32),
            pltpu.VMEM((NSLOT, CHUNK, DIM), jnp.float32),
            pltpu.VMEM((NSLOT, CHUNK, DIM), jnp.float32),
            pltpu.VMEM((bpw,), jnp.float32),
            pltpu.SemaphoreType.DMA((2, NSLOT)),
        ],
    )(user_idx.astype(jnp.int32), item_idx.astype(jnp.int32), W, H)
    return y.reshape(-1, 1)


# speedup vs baseline: 1.1202x; 1.1202x over previous
import functools

import jax
import jax.numpy as jnp
from jax import lax
from jax.experimental import pallas as pl
from jax.experimental.pallas import tpu as pltpu
from jax.experimental.pallas import tpu_sc as plsc

DIM = 128
LANES = 16
CHUNK = 32
NSTEP = DIM // LANES

_PERM_DNUMS = lax.GatherDimensionNumbers(
    offset_dims=(), collapsed_slice_dims=(0,), start_index_map=(0,))


def _lane_perm(x, idx):
    return lax.gather(x, idx[:, None], _PERM_DNUMS, (1,),
                      mode=lax.GatherScatterMode.PROMISE_IN_BOUNDS)


NSLOT = 4


def _bpr_body(user_hbm, item_hbm, w_hbm, h_hbm, out_hbm,
              uidx_v, iidx_v, wbufs, hbufs, out_v, sems, *,
              num_cores, bpw):
    wid = lax.axis_index("s") * num_cores + lax.axis_index("c")
    nchunk = bpw // CHUNK
    nsuper = nchunk // NSLOT
    base = wid * bpw

    pltpu.sync_copy(user_hbm.at[pl.ds(base, bpw)], uidx_v)
    pltpu.sync_copy(item_hbm.at[pl.ds(base, bpw)], iidx_v)

    lane_iota = lax.iota(jnp.int32, LANES)

    def fire(c, j):
        off = pl.multiple_of(c * CHUNK, CHUNK)
        pltpu.async_copy(w_hbm.at[uidx_v.at[pl.ds(off, CHUNK)]],
                         wbufs.at[j], sems.at[0, j])
        pltpu.async_copy(h_hbm.at[iidx_v.at[pl.ds(off, CHUNK)]],
                         hbufs.at[j], sems.at[1, j])

    def wait(j):
        pltpu.make_async_copy(w_hbm.at[uidx_v.at[pl.ds(0, CHUNK)]],
                              wbufs.at[j], sems.at[0, j]).wait()
        pltpu.make_async_copy(h_hbm.at[iidx_v.at[pl.ds(0, CHUNK)]],
                              hbufs.at[j], sems.at[1, j]).wait()

    def compute(c, j):
        wbuf = wbufs.at[j]
        hbuf = hbufs.at[j]
        def row_body(row, res):
            acc = (wbuf[row, pl.ds(0, LANES)]
                   * hbuf[row, pl.ds(0, LANES)])
            for k in range(1, NSTEP):
                acc = acc + (wbuf[row, pl.ds(k * LANES, LANES)]
                             * hbuf[row, pl.ds(k * LANES, LANES)])
            for sh in (8, 4, 2, 1):
                acc = acc + _lane_perm(acc, lane_iota ^ sh)
            res = jnp.where(lane_iota == (row & (LANES - 1)), acc, res)
            @pl.when((row & (LANES - 1)) == LANES - 1)
            def _():
                out_v[pl.ds(c * CHUNK + (row & -LANES), LANES)] = res
            return res

        lax.fori_loop(0, CHUNK, row_body,
                      jnp.zeros((LANES,), jnp.float32), unroll=4)

    for j in range(NSLOT):
        fire(j, j)

    def super_body(t, carry):
        c0 = t * NSLOT
        for j in range(NSLOT):
            wait(j)
            compute(c0 + j, j)

            @pl.when(t + 1 < nsuper)
            def _(j=j):
                fire(c0 + j + NSLOT, j)
        return carry

    lax.fori_loop(0, nsuper, super_body, 0)

    pltpu.sync_copy(out_v, out_hbm.at[pl.ds(base, bpw)])


def kernel(user_idx, item_idx, W, H):
    batch = user_idx.shape[0]
    info = plsc.get_sparse_core_info()
    num_workers = info.num_cores * info.num_subcores
    bpw = batch // num_workers

    mesh = plsc.VectorSubcoreMesh(core_axis_name="c", subcore_axis_name="s")
    body = functools.partial(_bpr_body, num_cores=info.num_cores, bpw=bpw)

    y = pl.kernel(
        body,
        mesh=mesh,
        out_type=jax.ShapeDtypeStruct((batch,), jnp.float32),
        scratch_types=[
            pltpu.VMEM((bpw,), jnp.int32),
            pltpu.VMEM((bpw,), jnp.int32),
            pltpu.VMEM((NSLOT, CHUNK, DIM), jnp.float32),
            pltpu.VMEM((NSLOT, CHUNK, DIM), jnp.float32),
            pltpu.VMEM((bpw,), jnp.float32),
            pltpu.SemaphoreType.DMA((2, NSLOT)),
        ],
    )(user_idx.astype(jnp.int32), item_idx.astype(jnp.int32), W, H)
    return y.reshape(-1, 1)
